# baseline (speedup 1.0000x reference)
import jax
import jax.numpy as jnp
from jax import lax
from jax.experimental import pallas as pl
from jax.experimental.pallas import tpu as pltpu
from jax.experimental.pallas import tpu_sc as plsc

N = 10000
D = 128
E = 320000
R = 10240
TILES = 32
EPT = R
EP = TILES * EPT
K = 64
CH = EPT // K
NBI = 4
NBS = 2
NGRP = CH // NBI
STRIPE = R // 16
GB = 8
BR = R // GB
MASK_RATIO = 0.1
NOISE_STD = 0.1

_mesh = plsc.VectorSubcoreMesh(core_axis_name="c", subcore_axis_name="s")
_sc_params = pltpu.CompilerParams(needs_layout_passes=False,
                                  use_tc_tiling_on_sc=False)


HR = 2 * R // D


def _sc_hist(pk_hbm, zr_hbm, out_hbm, pk_v, hv, ibuf, shacc):
    c = lax.axis_index("c")
    s = lax.axis_index("s")
    wid = c * 16 + s
    pltpu.sync_copy(pk_hbm.at[wid], pk_v)
    pltpu.sync_copy(zr_hbm.at[pl.ds(0, HR // 16)],
                    shacc.at[pl.ds(s * (HR // 16), HR // 16)])
    zero = jnp.zeros((16,), jnp.float32)

    def zbody(i, carry):
        hv[i, pl.ds(0, 16)] = zero
        hv[i, pl.ds(16, 16)] = zero
        hv[i, pl.ds(32, 16)] = zero
        hv[i, pl.ds(48, 16)] = zero
        hv[i, pl.ds(64, 16)] = zero
        hv[i, pl.ds(80, 16)] = zero
        hv[i, pl.ds(96, 16)] = zero
        hv[i, pl.ds(112, 16)] = zero
        return carry

    lax.fori_loop(0, HR, zbody, 0)
    iota = jnp.arange(16, dtype=jnp.int32)

    def ibody(i, carry):
        ibuf[pl.ds(i * 16, 16)] = iota + i * 16
        return carry

    lax.fori_loop(0, HR // 16, ibody, 0)
    one = jnp.ones((16,), jnp.float32)

    def ubody(i, carry):
        v = pk_v[i, :]
        lo = jnp.bitwise_and(v, 65535)
        hi = lax.shift_right_logical(v, 16)
        plsc.addupdate_scatter(hv, [lax.shift_right_logical(lo, 7),
                                    jnp.bitwise_and(lo, 127)], one)
        plsc.addupdate_scatter(hv, [lax.shift_right_logical(hi, 7) + (R // D),
                                    jnp.bitwise_and(hi, 127)], one)
        return carry

    lax.fori_loop(0, EPT // 16, ubody, 0)
    plsc.subcore_barrier()
    pltpu.sync_copy(hv, shacc.at[ibuf], add=True)
    plsc.subcore_barrier()
    pltpu.sync_copy(shacc.at[pl.ds(s * (HR // 16), HR // 16)],
                    out_hbm.at[c, pl.ds(s * (HR // 16), HR // 16)])


_hist_call = pl.kernel(
    _sc_hist,
    out_type=jax.ShapeDtypeStruct((2, HR, D), jnp.float32),
    mesh=_mesh,
    compiler_params=_sc_params,
    scratch_types=[
        pltpu.VMEM((EPT // 16, 16), jnp.int32),
        pltpu.VMEM((HR, D), jnp.float32),
        pltpu.VMEM((HR,), jnp.int32),
        pltpu.VMEM_SHARED((HR, D), jnp.float32),
    ],
)


def _sc_agg(xs_hbm, pk_hbm, zr_hbm, out_hbm, acc, pk_v, idxb, idxs, rbf, rf,
            gsem, ssem):
    c = lax.axis_index("c")
    s = lax.axis_index("s")
    wid = c * 16 + s
    pltpu.sync_copy(zr_hbm, acc.at[pl.ds(s * STRIPE, STRIPE)])
    pltpu.sync_copy(pk_hbm.at[wid], pk_v)
    plsc.subcore_barrier()

    def unpack_src(j, q):
        for t in range(K // 16):
            v = pk_v[j, pl.ds(t * 16, 16)]
            idxb[q, pl.ds(t * 16, 16)] = jnp.bitwise_and(v, 65535)

    def unpack_dst(j, qs):
        for t in range(K // 16):
            v = pk_v[j, pl.ds(t * 16, 16)]
            idxs[qs, pl.ds(t * 16, 16)] = lax.shift_right_logical(v, 16)

    def convert(q, qs):
        def cbody(k, carry):
            for t in range(D // 32):
                v = rbf[q, k, pl.ds(t * 16, 16)]
                e = plsc.bitcast(lax.shift_left(v, 16), jnp.float32)
                o = plsc.bitcast(jnp.bitwise_and(v, -65536), jnp.float32)
                rf[qs, k, pl.ds(t * 16, 16)] = e
                rf[qs, k, pl.ds(64 + t * 16, 16)] = o
            return carry

        lax.fori_loop(0, K, cbody, 0)

    def start_g(q):
        pltpu.async_copy(xs_hbm.at[idxb.at[q]], rbf.at[q], gsem.at[q])

    def wait_g(q):
        pltpu.make_async_copy(xs_hbm.at[pl.ds(0, K)], rbf.at[q],
                              gsem.at[q]).wait()

    def start_s(qs):
        pltpu.async_copy(rf.at[qs], acc.at[idxs.at[qs]], ssem.at[qs],
                         add=True)

    def wait_s(qs):
        pltpu.make_async_copy(rf.at[qs], acc.at[pl.ds(0, K)],
                              ssem.at[qs]).wait()

    for q in range(NBI):
        unpack_src(q, q)
        start_g(q)

    def body(g, carry):
        for q in range(NBI):
            j = g * NBI + q
            qs = q % NBS
            wait_g(q)
            if q < NBS:
                @pl.when(g > 0)
                def _():
                    wait_s(qs)
            else:
                wait_s(qs)
            unpack_dst(j, qs)
            convert(q, qs)

            @pl.when(g + 1 < NGRP)
            def _():
                unpack_src(j + NBI, q)
                start_g(q)

            start_s(qs)

        return carry

    lax.fori_loop(0, NGRP, body, 0)
    for qs in range(NBS):
        wait_s(qs)
    plsc.subcore_barrier()
    pltpu.sync_copy(acc.at[pl.ds(s * STRIPE, STRIPE)],
                    out_hbm.at[c, pl.ds(s * STRIPE, STRIPE)])


_agg_call = pl.kernel(
    _sc_agg,
    out_type=jax.ShapeDtypeStruct((2, R, D), jnp.float32),
    mesh=_mesh,
    compiler_params=_sc_params,
    scratch_types=[
        pltpu.VMEM_SHARED((R, D), jnp.float32),
        pltpu.VMEM((CH, K), jnp.int32),
        pltpu.VMEM((NBI, K), jnp.int32),
        pltpu.VMEM((NBS, K), jnp.int32),
        pltpu.VMEM((NBI, K, D // 2), jnp.int32),
        pltpu.VMEM((NBS, K, D), jnp.float32),
        pltpu.SemaphoreType.DMA((NBI,)),
        pltpu.SemaphoreType.DMA((NBS,)),
    ],
)


def _scales(degs):
    so = lax.rsqrt(jnp.maximum(degs[:, 0:1], 1.0))
    si = lax.rsqrt(jnp.maximum(degs[:, 1:2], 1.0))
    return so, si


def _tc_prep(x_ref, mb_ref, nz_ref, tok_ref, degT_ref, xs_ref):
    so, _ = _scales(degT_ref[...])
    xv = x_ref[...]
    xm = mb_ref[...] * (tok_ref[...] - xv) + xv + nz_ref[...]
    xs_ref[...] = (xm * so).astype(jnp.bfloat16)


def _layernorm(h, g, b):
    mu = jnp.mean(h, axis=1, keepdims=True)
    var = jnp.mean((h - mu) ** 2, axis=1, keepdims=True)
    return (h - mu) * lax.rsqrt(var + 1e-5) * g + b


def _tc_layer1(p_ref, degT_ref, w_ref, b_ref, g_ref, be_ref, o_ref):
    so, si = _scales(degT_ref[...])
    agg = (p_ref[0] + p_ref[1]) * si
    h = jnp.dot(agg, w_ref[...], preferred_element_type=jnp.float32,
                precision=lax.Precision.HIGHEST) + b_ref[...]
    h = _layernorm(h, g_ref[...], be_ref[...])
    o_ref[...] = (jnp.maximum(h, 0.0) * so).astype(jnp.bfloat16)


def _tc_final(p_ref, degT_ref, w2_ref, b2_ref, g2_ref, be2_ref,
              wd1_ref, bd1_ref, wd2_ref, bd2_ref, z_ref, xr_ref):
    _, si = _scales(degT_ref[...])
    agg = (p_ref[0] + p_ref[1]) * si
    h = jnp.dot(agg, w2_ref[...], preferred_element_type=jnp.float32,
                precision=lax.Precision.HIGHEST) + b2_ref[...]
    z = _layernorm(h, g2_ref[...], be2_ref[...])
    z_ref[...] = z
    hd = jnp.maximum(jnp.dot(z, wd1_ref[...], preferred_element_type=jnp.float32,
                             precision=lax.Precision.HIGHEST) + bd1_ref[...], 0.0)
    xr_ref[...] = jnp.dot(hd, wd2_ref[...], preferred_element_type=jnp.float32,
                          precision=lax.Precision.HIGHEST) + bd2_ref[...]


def _row_spec():
    return pl.BlockSpec((BR, D), lambda i: (i, 0))


def _vec_spec():
    return pl.BlockSpec((1, D), lambda i: (0, 0))


def _mat_spec():
    return pl.BlockSpec((D, D), lambda i: (0, 0))


def _swz(w):
    return jnp.concatenate([w[0::2], w[1::2]], axis=0)


def kernel(x, edge_index, mask_token, W1, b1, g1, be1, W2, b2, g2, be2,
           Wd1, bd1, Wd2, bd2):
    f32 = jnp.float32
    with jax.ensure_compile_time_eval():
        num_mask = max(1, int(MASK_RATIO * N))
        perm = jax.random.permutation(jax.random.key(1), N)
        mask_idx = perm[:num_mask]
        node_mask = jnp.zeros((N,), dtype=bool).at[mask_idx].set(True)
        noise = jax.random.normal(jax.random.key(2), (N, D), dtype=f32) * NOISE_STD
        maskb = jnp.pad(jnp.broadcast_to(node_mask[:, None], (N, D)).astype(f32),
                        ((0, R - N), (0, 0)))
        noise_p = jnp.pad(noise, ((0, R - N), (0, 0)))
        padv = jnp.full((EP - E,), N + N * 65536, jnp.int32)
        zrow = jnp.zeros((STRIPE, D), f32)

    tok = mask_token[None, :]

    x_p = jnp.pad(x, ((0, R - N), (0, 0)))
    packed = jnp.concatenate([edge_index[0] + edge_index[1] * 65536, padv])
    pk_h = packed.reshape(TILES, EPT // 16, 16)
    pk_a = packed.reshape(TILES, CH, K)

    hist = _hist_call(pk_h, zrow)
    ht = hist[0] + hist[1]
    degT = jnp.pad(jnp.stack([ht[:R // D].reshape(R), ht[R // D:].reshape(R)],
                             axis=1), ((0, 0), (0, 6)))

    xs = pl.pallas_call(
        _tc_prep,
        grid=(GB,),
        in_specs=[_row_spec(), _row_spec(), _row_spec(), _vec_spec(),
                  pl.BlockSpec((BR, 8), lambda i: (i, 0))],
        out_specs=_row_spec(),
        out_shape=jax.ShapeDtypeStruct((R, D), jnp.bfloat16),
    )(x_p, maskb, noise_p, tok, degT)

    xs_i = lax.bitcast_convert_type(xs.reshape(R, D // 2, 2), jnp.int32)
    p1 = _agg_call(xs_i, pk_a, zrow)

    xs2 = pl.pallas_call(
        _tc_layer1,
        grid=(GB,),
        in_specs=[pl.BlockSpec((2, BR, D), lambda i: (0, i, 0)),
                  pl.BlockSpec((BR, 8), lambda i: (i, 0)), _mat_spec(),
                  _vec_spec(), _vec_spec(), _vec_spec()],
        out_specs=_row_spec(),
        out_shape=jax.ShapeDtypeStruct((R, D), jnp.bfloat16),
    )(p1, degT, _swz(W1), b1[None, :], g1[None, :], be1[None, :])

    xs2_i = lax.bitcast_convert_type(xs2.reshape(R, D // 2, 2), jnp.int32)
    p2 = _agg_call(xs2_i, pk_a, zrow)

    z_pad, xr_pad = pl.pallas_call(
        _tc_final,
        grid=(GB,),
        in_specs=[pl.BlockSpec((2, BR, D), lambda i: (0, i, 0)),
                  pl.BlockSpec((BR, 8), lambda i: (i, 0)), _mat_spec(),
                  _vec_spec(), _vec_spec(),
                  _vec_spec(), _mat_spec(), _vec_spec(), _mat_spec(),
                  _vec_spec()],
        out_specs=[_row_spec(), _row_spec()],
        out_shape=[jax.ShapeDtypeStruct((R, D), f32)] * 2,
    )(p2, degT, _swz(W2), b2[None, :], g2[None, :], be2[None, :],
      Wd1, bd1[None, :], Wd2, bd2[None, :])

    return (xr_pad[:N], x, node_mask, z_pad[:N])


# speedup vs baseline: 1.0456x; 1.0456x over previous
import jax
import jax.numpy as jnp
from jax import lax
from jax.experimental import pallas as pl
from jax.experimental.pallas import tpu as pltpu
from jax.experimental.pallas import tpu_sc as plsc

N = 10000
D = 128
E = 320000
R = 10240
TILES = 32
EPT = R
EP = TILES * EPT
K = 16
CH = EPT // K
NBI = 20
NBS = 2
NGRP = CH // NBI
STRIPE = R // 16
GB = 8
BR = R // GB
MASK_RATIO = 0.1
NOISE_STD = 0.1

_mesh = plsc.VectorSubcoreMesh(core_axis_name="c", subcore_axis_name="s")
_sc_params = pltpu.CompilerParams(needs_layout_passes=False,
                                  use_tc_tiling_on_sc=False)


HR = 2 * R // D


def _sc_hist(pk_hbm, zr_hbm, out_hbm, pk_v, hv, ibuf, shacc):
    c = lax.axis_index("c")
    s = lax.axis_index("s")
    wid = c * 16 + s
    pltpu.sync_copy(pk_hbm.at[wid], pk_v)
    pltpu.sync_copy(zr_hbm.at[pl.ds(0, HR // 16)],
                    shacc.at[pl.ds(s * (HR // 16), HR // 16)])
    zero = jnp.zeros((16,), jnp.float32)

    def zbody(i, carry):
        hv[i, pl.ds(0, 16)] = zero
        hv[i, pl.ds(16, 16)] = zero
        hv[i, pl.ds(32, 16)] = zero
        hv[i, pl.ds(48, 16)] = zero
        hv[i, pl.ds(64, 16)] = zero
        hv[i, pl.ds(80, 16)] = zero
        hv[i, pl.ds(96, 16)] = zero
        hv[i, pl.ds(112, 16)] = zero
        return carry

    lax.fori_loop(0, HR, zbody, 0)
    iota = jnp.arange(16, dtype=jnp.int32)

    def ibody(i, carry):
        ibuf[pl.ds(i * 16, 16)] = iota + i * 16
        return carry

    lax.fori_loop(0, HR // 16, ibody, 0)
    one = jnp.ones((16,), jnp.float32)

    def ubody(i, carry):
        v = pk_v[i, :]
        lo = jnp.bitwise_and(v, 65535)
        hi = lax.shift_right_logical(v, 16)
        plsc.addupdate_scatter(hv, [lax.shift_right_logical(lo, 7),
                                    jnp.bitwise_and(lo, 127)], one)
        plsc.addupdate_scatter(hv, [lax.shift_right_logical(hi, 7) + (R // D),
                                    jnp.bitwise_and(hi, 127)], one)
        return carry

    lax.fori_loop(0, EPT // 16, ubody, 0)
    plsc.subcore_barrier()
    pltpu.sync_copy(hv, shacc.at[ibuf], add=True)
    plsc.subcore_barrier()
    pltpu.sync_copy(shacc.at[pl.ds(s * (HR // 16), HR // 16)],
                    out_hbm.at[c, pl.ds(s * (HR // 16), HR // 16)])


_hist_call = pl.kernel(
    _sc_hist,
    out_type=jax.ShapeDtypeStruct((2, HR, D), jnp.float32),
    mesh=_mesh,
    compiler_params=_sc_params,
    scratch_types=[
        pltpu.VMEM((EPT // 16, 16), jnp.int32),
        pltpu.VMEM((HR, D), jnp.float32),
        pltpu.VMEM((HR,), jnp.int32),
        pltpu.VMEM_SHARED((HR, D), jnp.float32),
    ],
)


def _sc_agg(xs_hbm, pk_hbm, zr_hbm, out_hbm, acc, pk_v, idxb, idxs, rbf, rf,
            gsem, ssem):
    c = lax.axis_index("c")
    s = lax.axis_index("s")
    wid = c * 16 + s
    pltpu.sync_copy(zr_hbm, acc.at[pl.ds(s * STRIPE, STRIPE)])
    pltpu.sync_copy(pk_hbm.at[wid], pk_v)
    plsc.subcore_barrier()

    def unpack_src(j, q):
        for t in range(K // 16):
            v = pk_v[j, pl.ds(t * 16, 16)]
            idxb[q, pl.ds(t * 16, 16)] = jnp.bitwise_and(v, 65535)

    def unpack_dst(j, qs):
        for t in range(K // 16):
            v = pk_v[j, pl.ds(t * 16, 16)]
            idxs[qs, pl.ds(t * 16, 16)] = lax.shift_right_logical(v, 16)

    def convert(q, qs):
        def cbody(k, carry):
            for t in range(D // 32):
                v = rbf[q, k, pl.ds(t * 16, 16)]
                e = plsc.bitcast(lax.shift_left(v, 16), jnp.float32)
                o = plsc.bitcast(jnp.bitwise_and(v, -65536), jnp.float32)
                rf[qs, k, pl.ds(t * 16, 16)] = e
                rf[qs, k, pl.ds(64 + t * 16, 16)] = o
            return carry

        lax.fori_loop(0, K, cbody, 0)

    def start_g(q):
        pltpu.async_copy(xs_hbm.at[idxb.at[q]], rbf.at[q], gsem.at[q])

    def wait_g(q):
        pltpu.make_async_copy(xs_hbm.at[pl.ds(0, K)], rbf.at[q],
                              gsem.at[q]).wait()

    def start_s(qs):
        pltpu.async_copy(rf.at[qs], acc.at[idxs.at[qs]], ssem.at[qs],
                         add=True)

    def wait_s(qs):
        pltpu.make_async_copy(rf.at[qs], acc.at[pl.ds(0, K)],
                              ssem.at[qs]).wait()

    for q in range(NBI):
        unpack_src(q, q)
        start_g(q)

    def body(g, carry):
        for q in range(NBI):
            j = g * NBI + q
            qs = q % NBS
            wait_g(q)
            if q < NBS:
                @pl.when(g > 0)
                def _():
                    wait_s(qs)
            else:
                wait_s(qs)
            unpack_dst(j, qs)
            convert(q, qs)

            @pl.when(g + 1 < NGRP)
            def _():
                unpack_src(j + NBI, q)
                start_g(q)

            start_s(qs)

        return carry

    lax.fori_loop(0, NGRP, body, 0)
    for qs in range(NBS):
        wait_s(qs)
    plsc.subcore_barrier()
    pltpu.sync_copy(acc.at[pl.ds(s * STRIPE, STRIPE)],
                    out_hbm.at[c, pl.ds(s * STRIPE, STRIPE)])


_agg_call = pl.kernel(
    _sc_agg,
    out_type=jax.ShapeDtypeStruct((2, R, D), jnp.float32),
    mesh=_mesh,
    compiler_params=_sc_params,
    scratch_types=[
        pltpu.VMEM_SHARED((R, D), jnp.float32),
        pltpu.VMEM((CH, K), jnp.int32),
        pltpu.VMEM((NBI, K), jnp.int32),
        pltpu.VMEM((NBS, K), jnp.int32),
        pltpu.VMEM((NBI, K, D // 2), jnp.int32),
        pltpu.VMEM((NBS, K, D), jnp.float32),
        pltpu.SemaphoreType.DMA((NBI,)),
        pltpu.SemaphoreType.DMA((NBS,)),
    ],
)


def _scales(degs):
    so = lax.rsqrt(jnp.maximum(degs[:, 0:1], 1.0))
    si = lax.rsqrt(jnp.maximum(degs[:, 1:2], 1.0))
    return so, si


def _tc_prep(x_ref, mb_ref, nz_ref, tok_ref, degT_ref, xs_ref):
    so, _ = _scales(degT_ref[...])
    xv = x_ref[...]
    xm = mb_ref[...] * (tok_ref[...] - xv) + xv + nz_ref[...]
    xs_ref[...] = (xm * so).astype(jnp.bfloat16)


def _layernorm(h, g, b):
    mu = jnp.mean(h, axis=1, keepdims=True)
    var = jnp.mean((h - mu) ** 2, axis=1, keepdims=True)
    return (h - mu) * lax.rsqrt(var + 1e-5) * g + b


def _tc_layer1(p_ref, degT_ref, w_ref, b_ref, g_ref, be_ref, o_ref):
    so, si = _scales(degT_ref[...])
    agg = (p_ref[0] + p_ref[1]) * si
    h = jnp.dot(agg, w_ref[...], preferred_element_type=jnp.float32,
                precision=lax.Precision.HIGHEST) + b_ref[...]
    h = _layernorm(h, g_ref[...], be_ref[...])
    o_ref[...] = (jnp.maximum(h, 0.0) * so).astype(jnp.bfloat16)


def _tc_final(p_ref, degT_ref, w2_ref, b2_ref, g2_ref, be2_ref,
              wd1_ref, bd1_ref, wd2_ref, bd2_ref, z_ref, xr_ref):
    _, si = _scales(degT_ref[...])
    agg = (p_ref[0] + p_ref[1]) * si
    h = jnp.dot(agg, w2_ref[...], preferred_element_type=jnp.float32,
                precision=lax.Precision.HIGHEST) + b2_ref[...]
    z = _layernorm(h, g2_ref[...], be2_ref[...])
    z_ref[...] = z
    hd = jnp.maximum(jnp.dot(z, wd1_ref[...], preferred_element_type=jnp.float32,
                             precision=lax.Precision.HIGHEST) + bd1_ref[...], 0.0)
    xr_ref[...] = jnp.dot(hd, wd2_ref[...], preferred_element_type=jnp.float32,
                          precision=lax.Precision.HIGHEST) + bd2_ref[...]


def _row_spec():
    return pl.BlockSpec((BR, D), lambda i: (i, 0))


def _vec_spec():
    return pl.BlockSpec((1, D), lambda i: (0, 0))


def _mat_spec():
    return pl.BlockSpec((D, D), lambda i: (0, 0))


def _swz(w):
    return jnp.concatenate([w[0::2], w[1::2]], axis=0)


def kernel(x, edge_index, mask_token, W1, b1, g1, be1, W2, b2, g2, be2,
           Wd1, bd1, Wd2, bd2):
    f32 = jnp.float32
    with jax.ensure_compile_time_eval():
        num_mask = max(1, int(MASK_RATIO * N))
        perm = jax.random.permutation(jax.random.key(1), N)
        mask_idx = perm[:num_mask]
        node_mask = jnp.zeros((N,), dtype=bool).at[mask_idx].set(True)
        noise = jax.random.normal(jax.random.key(2), (N, D), dtype=f32) * NOISE_STD
        maskb = jnp.pad(jnp.broadcast_to(node_mask[:, None], (N, D)).astype(f32),
                        ((0, R - N), (0, 0)))
        noise_p = jnp.pad(noise, ((0, R - N), (0, 0)))
        padv = jnp.full((EP - E,), N + N * 65536, jnp.int32)
        zrow = jnp.zeros((STRIPE, D), f32)

    tok = mask_token[None, :]

    x_p = jnp.pad(x, ((0, R - N), (0, 0)))
    packed = jnp.concatenate([edge_index[0] + edge_index[1] * 65536, padv])
    pk_h = packed.reshape(TILES, EPT // 16, 16)
    pk_a = packed.reshape(TILES, CH, K)

    hist = _hist_call(pk_h, zrow)
    ht = hist[0] + hist[1]
    degT = jnp.pad(jnp.stack([ht[:R // D].reshape(R), ht[R // D:].reshape(R)],
                             axis=1), ((0, 0), (0, 6)))

    xs = pl.pallas_call(
        _tc_prep,
        grid=(GB,),
        in_specs=[_row_spec(), _row_spec(), _row_spec(), _vec_spec(),
                  pl.BlockSpec((BR, 8), lambda i: (i, 0))],
        out_specs=_row_spec(),
        out_shape=jax.ShapeDtypeStruct((R, D), jnp.bfloat16),
    )(x_p, maskb, noise_p, tok, degT)

    xs_i = lax.bitcast_convert_type(xs.reshape(R, D // 2, 2), jnp.int32)
    p1 = _agg_call(xs_i, pk_a, zrow)

    xs2 = pl.pallas_call(
        _tc_layer1,
        grid=(GB,),
        in_specs=[pl.BlockSpec((2, BR, D), lambda i: (0, i, 0)),
                  pl.BlockSpec((BR, 8), lambda i: (i, 0)), _mat_spec(),
                  _vec_spec(), _vec_spec(), _vec_spec()],
        out_specs=_row_spec(),
        out_shape=jax.ShapeDtypeStruct((R, D), jnp.bfloat16),
    )(p1, degT, _swz(W1), b1[None, :], g1[None, :], be1[None, :])

    xs2_i = lax.bitcast_convert_type(xs2.reshape(R, D // 2, 2), jnp.int32)
    p2 = _agg_call(xs2_i, pk_a, zrow)

    z_pad, xr_pad = pl.pallas_call(
        _tc_final,
        grid=(GB,),
        in_specs=[pl.BlockSpec((2, BR, D), lambda i: (0, i, 0)),
                  pl.BlockSpec((BR, 8), lambda i: (i, 0)), _mat_spec(),
                  _vec_spec(), _vec_spec(),
                  _vec_spec(), _mat_spec(), _vec_spec(), _mat_spec(),
                  _vec_spec()],
        out_specs=[_row_spec(), _row_spec()],
        out_shape=[jax.ShapeDtypeStruct((R, D), f32)] * 2,
    )(p2, degT, _swz(W2), b2[None, :], g2[None, :], be2[None, :],
      Wd1, bd1[None, :], Wd2, bd2[None, :])

    return (xr_pad[:N], x, node_mask, z_pad[:N])
